# baseline (speedup 1.0000x reference)
import functools

import jax
import jax.numpy as jnp
from jax import lax
from jax.experimental import pallas as pl
from jax.experimental.pallas import tpu as pltpu
from jax.experimental.pallas import tpu_sc as plsc

N = 50000
E = 800000
IN_DIM = 128
H = 64
ED = 16
EMB = 512
G = 64

NP = 50176
BN_TC = 896
NBLK = NP // BN_TC
HNODE = NP // 2
HROWS = HNODE // 4
SROWS = 6400
E2 = 819200
NSUP = E2 // 16 // 1024
SDROW = E2 // 128
EOFF = E2 // 2
PT = NP // 32
PCH = 112
NPCH = PT // PCH

_mesh = plsc.VectorSubcoreMesh(core_axis_name="c", subcore_axis_name="s")


def _prep_body(x_ref, w_ref, b_ref, o_ref):
    h = jnp.dot(x_ref[...], w_ref[...], preferred_element_type=jnp.float32)
    h = h + b_ref[...]
    o_ref[...] = jnp.concatenate(
        [h, jnp.zeros((BN_TC, IN_DIM - H), jnp.float32)], axis=1)


def _eproj_body(ea_ref, w_ref, b_ref, o1_ref, o2_ref, o3_ref):
    e = jnp.dot(ea_ref[...], w_ref[...],
                preferred_element_type=jnp.float32) + b_ref[...]
    o1_ref[...] = e[:, 0:128]
    o2_ref[...] = e[:, 128:256]
    o3_ref[...] = e[:, 256:384]


def _mlp_body(hp_ref, ag_ref, w1_ref, b1_ref, w2_ref, b2_ref, s_ref, t_ref,
              o_ref):
    ag = ag_ref[...]
    a0 = ag[0]
    a1 = ag[1]
    hin = hp_ref[...][:, :H] + jnp.concatenate([a0, a1], axis=1)
    t1 = jnp.maximum(
        jnp.dot(hin, w1_ref[...], preferred_element_type=jnp.float32)
        + b1_ref[...], 0.0)
    t2 = jnp.dot(t1, w2_ref[...], preferred_element_type=jnp.float32) + b2_ref[...]
    y = jnp.maximum(t2 * s_ref[...] + t_ref[...], 0.0)
    o_ref[...] = jnp.concatenate(
        [y, jnp.zeros((BN_TC, IN_DIM - H), jnp.float32)], axis=1)


def _head_body(sa_ref, sm_ref, w_ref, b_ref, g_ref, t_ref, o_ref):
    sa = jnp.sum(sa_ref[...], axis=0)
    mx = jnp.max(sm_ref[...], axis=0)
    sums = sa[:G, :H]
    cnt = sa[:G, H:H + 1]
    mean = sums / jnp.maximum(cnt, 1.0)
    cat = jnp.concatenate([mean, mx[:G, :H]], axis=1)
    o = jnp.dot(cat, w_ref[...], preferred_element_type=jnp.float32) + b_ref[...]
    mu = jnp.mean(o, axis=-1, keepdims=True)
    var = jnp.mean((o - mu) ** 2, axis=-1, keepdims=True)
    o = (o - mu) / jnp.sqrt(var + 1e-5) * g_ref[...] + t_ref[...]
    o_ref[...] = jnp.maximum(o, 0.0)


@functools.partial(
    pl.kernel,
    mesh=_mesh,
    out_type=jax.ShapeDtypeStruct((2, 2, HROWS, 128), jnp.float32),
    scratch_types=[
        pltpu.VMEM((16,), jnp.int32),
        pltpu.VMEM((16, 128), jnp.float32),
        pltpu.VMEM((128,), jnp.int32),
        pltpu.VMEM((64,), jnp.int32),
        pltpu.VMEM((128,), jnp.int32),
        pltpu.VMEM((128,), jnp.int32),
        pltpu.VMEM((128, 128), jnp.float32),
        pltpu.VMEM((64, 128), jnp.float32),
        pltpu.VMEM((128, 128), jnp.float32),
        pltpu.VMEM_SHARED((SROWS, 128), jnp.float32),
        pltpu.SemaphoreType.DMA,
        pltpu.SemaphoreType.DMA,
        pltpu.SemaphoreType.DMA,
    ],
)
def _edge_pass(hp, ep, out, sdidx, sdbuf, srcv, epidx, rowv, selv, hbuf,
               ebuf, vbuf, shared, sem, sem2, sem3):
    cid = lax.axis_index("c")
    sid = lax.axis_index("s")
    z16 = jnp.zeros((16,), jnp.float32)
    iota = lax.iota(jnp.int32, 16)

    def zrow(i, _):
        for g in range(8):
            vbuf[i, pl.ds(g * 16, 16)] = z16
        return 0

    lax.fori_loop(0, 128, zrow, 0)

    fo = cid * 32

    for p in range(2):
        nbase = p * HNODE
        for k, (zo, zn) in enumerate(((0, 112), (112, 112), (224, 112),
                                      (336, 64))):
            pltpu.sync_copy(vbuf.at[pl.ds(0, zn)],
                            shared.at[pl.ds(sid * 400 + zo, zn)])
        plsc.subcore_barrier()

        def super_chunk(ss, _):
            g = sid * NSUP + ss
            sdidx[pl.ds(0, 16)] = (EOFF + g * 8
                                   + jnp.where(iota < 8, iota, iota + SDROW - 8))
            pltpu.async_copy(ep.at[sdidx], sdbuf, sem3).wait()

            def sub_chunk(sc, _):
                ebase = g * 512 + sc * 64
                for k in range(4):
                    epidx[pl.ds(k * 16, 16)] = ebase + k * 16 + iota
                for k in range(8):
                    srcv[pl.ds(k * 16, 16)] = sdbuf[
                        sc, pl.ds(k * 16, 16)].astype(jnp.int32)
                gath = pltpu.async_copy(hp.at[srcv], hbuf, sem)
                egath = pltpu.async_copy(ep.at[epidx], ebuf, sem2)
                for k in range(8):
                    d = sdbuf[8 + sc, pl.ds(k * 16, 16)].astype(jnp.int32)
                    dd = d - nbase
                    valid = jnp.logical_and(dd >= 0, dd < HNODE)
                    rowv[pl.ds(k * 16, 16)] = jnp.where(
                        valid, jnp.right_shift(dd, 2), HROWS + 16)
                    selv[pl.ds(k * 16, 16)] = jnp.bitwise_and(d, 3)
                gath.wait()
                egath.wait()
                for j in range(128):
                    sel = selv[pl.ds((j // 16) * 16, 16)][j % 16]
                    co = sel * 32
                    eb = (j % 2) * 64 + fo
                    for gg in range(2):
                        hv = hbuf[j, pl.ds(fo + gg * 16, 16)]
                        ev = ebuf[j // 2, pl.ds(eb + gg * 16, 16)]
                        vbuf[j, pl.ds(co + gg * 16, 16)] = jnp.maximum(
                            hv + ev, 0.0)
                pltpu.sync_copy(vbuf, shared.at[rowv], add=True)
                for j in range(128):
                    sel = selv[pl.ds((j // 16) * 16, 16)][j % 16]
                    co = sel * 32
                    vbuf[j, pl.ds(co, 16)] = z16
                    vbuf[j, pl.ds(co + 16, 16)] = z16
                return 0

            lax.fori_loop(0, 8, sub_chunk, 0)
            return 0

        lax.fori_loop(0, NSUP, super_chunk, 0)
        plsc.subcore_barrier()

        for k, (zo, zn) in enumerate(((0, 112), (112, 112), (224, 112),
                                      (336, 56))):
            off = sid * 392 + zo
            pltpu.sync_copy(shared.at[pl.ds(off, zn)],
                            out.at[cid, p, pl.ds(off, zn)])
        plsc.subcore_barrier()


@functools.partial(
    pl.kernel,
    mesh=_mesh,
    out_type=(
        jax.ShapeDtypeStruct((32, 72, 128), jnp.float32),
        jax.ShapeDtypeStruct((32, 72, 128), jnp.float32),
    ),
    scratch_types=[
        pltpu.VMEM((PCH,), jnp.int32),
        pltpu.VMEM((PCH, 128), jnp.float32),
        pltpu.VMEM((72, 128), jnp.float32),
        pltpu.VMEM((72, 128), jnp.float32),
        pltpu.SemaphoreType.DMA,
    ],
)
def _pool(hp, batch, out_a, out_m, bv, hbuf, acc_a, acc_m, sem):
    cid = lax.axis_index("c")
    sid = lax.axis_index("s")
    wid = sid * 2 + cid
    z16 = jnp.zeros((16,), jnp.float32)
    ninf = jnp.full((16,), -jnp.inf, jnp.float32)
    one0 = jnp.where(lax.iota(jnp.int32, 16) == 0, 1.0, 0.0).astype(jnp.float32)

    def zrow(i, _):
        for g in range(8):
            acc_a[i, pl.ds(g * 16, 16)] = z16
            acc_m[i, pl.ds(g * 16, 16)] = ninf
        return 0

    lax.fori_loop(0, 72, zrow, 0)

    base = wid * PT

    def chunk(ci, _):
        off = base + ci * PCH
        pltpu.sync_copy(batch.at[pl.ds(off, PCH)], bv)
        pltpu.sync_copy(hp.at[pl.ds(off, PCH)], hbuf)
        for j in range(PCH):
            b = bv[pl.ds((j // 16) * 16, 16)][j % 16]
            for g in range(4):
                s = pl.ds(g * 16, 16)
                acc_a[b, s] += hbuf[j, s]
                acc_m[b, s] = jnp.maximum(acc_m[b, s], hbuf[j, s])
            acc_a[b, pl.ds(64, 16)] += one0
        return 0

    lax.fori_loop(0, NPCH, chunk, 0)
    pltpu.sync_copy(acc_a, out_a.at[wid])
    pltpu.sync_copy(acc_m, out_m.at[wid])


def _tc_call(body, grid, in_specs, out_specs, out_shape):
    return pl.pallas_call(
        body, grid=grid, in_specs=in_specs, out_specs=out_specs,
        out_shape=out_shape)


def kernel(x, edge_index, edge_attr, batch, np_W, np_b, lin1_W, lin1_b,
           mlp1_W1, mlp1_b1, mlp1_W2, mlp1_b2, bn1_g, bn1_b, lin2_W, lin2_b,
           mlp2_W1, mlp2_b1, mlp2_W2, mlp2_b2, bn2_g, bn2_b, lin3_W, lin3_b,
           mlp3_W1, mlp3_b1, mlp3_W2, mlp3_b2, bn3_g, bn3_b, proj_W, proj_b,
           ln_g, ln_b):
    f32 = jnp.float32
    srcp = jnp.concatenate(
        [edge_index[0], jnp.zeros((E2 - E,), jnp.int32)])
    dstp = jnp.concatenate(
        [edge_index[1], jnp.full((E2 - E,), NP - 1, jnp.int32)])
    sd_rows = jnp.concatenate(
        [srcp, dstp]).reshape(2 * SDROW, 128).astype(jnp.float32)
    x_pad = jnp.zeros((NP, IN_DIM), f32).at[:N].set(x)
    batch_pad = jnp.concatenate(
        [batch, jnp.full((NP - N,), G, jnp.int32)])
    ea32 = edge_attr.reshape(E // 2, 2 * ED)

    wfull = lambda shp: pl.BlockSpec(shp, lambda i: (0,) * len(shp))

    hp = _tc_call(
        _prep_body, (NBLK,),
        [pl.BlockSpec((BN_TC, IN_DIM), lambda i: (i, 0)),
         wfull((IN_DIM, H)), wfull((1, H))],
        pl.BlockSpec((BN_TC, IN_DIM), lambda i: (i, 0)),
        jax.ShapeDtypeStruct((NP, IN_DIM), f32),
    )(x_pad, np_W, np_b.reshape(1, H))

    zw = jnp.zeros((ED, H), f32)
    wbd = jnp.block([
        [lin1_W, zw, lin2_W, zw, lin3_W, zw],
        [zw, lin1_W, zw, lin2_W, zw, lin3_W],
    ])
    bbd = jnp.concatenate(
        [lin1_b, lin1_b, lin2_b, lin2_b, lin3_b, lin3_b]).reshape(1, 6 * H)
    eshape = jax.ShapeDtypeStruct((E // 2, 128), f32)
    e1, e2, e3 = _tc_call(
        _eproj_body, (E // 8000,),
        [pl.BlockSpec((4000, 2 * ED), lambda i: (i, 0)),
         wfull((2 * ED, 6 * H)), wfull((1, 6 * H))],
        [pl.BlockSpec((4000, 128), lambda i: (i, 0))] * 3,
        [eshape, eshape, eshape],
    )(ea32, wbd, bbd)

    mlp_in_specs = [
        pl.BlockSpec((BN_TC, IN_DIM), lambda i: (i, 0)),
        pl.BlockSpec((2, BN_TC, 32), lambda i: (0, i, 0)),
        wfull((H, H)), wfull((1, H)), wfull((H, H)), wfull((1, H)),
        wfull((1, H)), wfull((1, H)),
    ]
    bn_s = 1.0 / jnp.sqrt(jnp.float32(1.0 + 1e-5))

    for (ep, w1, b1, w2, b2, bg, bb) in (
        (e1, mlp1_W1, mlp1_b1, mlp1_W2, mlp1_b2, bn1_g, bn1_b),
        (e2, mlp2_W1, mlp2_b1, mlp2_W2, mlp2_b2, bn2_g, bn2_b),
        (e3, mlp3_W1, mlp3_b1, mlp3_W2, mlp3_b2, bn3_g, bn3_b),
    ):
        combo = jnp.concatenate(
            [ep, jnp.zeros(((E2 - E) // 2, 128), f32), sd_rows])
        aggr = _edge_pass(hp, combo).reshape(2, NP, 32)
        hp = _tc_call(
            _mlp_body, (NBLK,),
            mlp_in_specs,
            pl.BlockSpec((BN_TC, IN_DIM), lambda i: (i, 0)),
            jax.ShapeDtypeStruct((NP, IN_DIM), f32),
        )(hp, aggr, w1, b1.reshape(1, H), w2, b2.reshape(1, H),
          (bg * bn_s).reshape(1, H), bb.reshape(1, H))

    sa, sm = _pool(hp, batch_pad)

    out = _tc_call(
        _head_body, (1,),
        [wfull((32, 72, 128)), wfull((32, 72, 128)),
         wfull((2 * H, EMB)), wfull((1, EMB)), wfull((1, EMB)),
         wfull((1, EMB))],
        wfull((G, EMB)),
        jax.ShapeDtypeStruct((G, EMB), f32),
    )(sa, sm, proj_W, proj_b.reshape(1, EMB), ln_g.reshape(1, EMB),
      ln_b.reshape(1, EMB))
    return out


# speedup vs baseline: 1.1596x; 1.1596x over previous
import functools

import jax
import jax.numpy as jnp
from jax import lax
from jax.experimental import pallas as pl
from jax.experimental.pallas import tpu as pltpu
from jax.experimental.pallas import tpu_sc as plsc

N = 50000
E = 800000
IN_DIM = 128
H = 64
ED = 16
EMB = 512
G = 64

NP = 50176
BN_TC = 896
NBLK = NP // BN_TC
HNODE = NP // 2
HROWS = HNODE // 4
SROWS = 6400
E2 = 819200
NSUP = E2 // 16 // 1024
SDROW = E2 // 128
EOFF = E2 // 2
PT = NP // 32
PCH = 112
NPCH = PT // PCH

_mesh = plsc.VectorSubcoreMesh(core_axis_name="c", subcore_axis_name="s")


def _prep_body(x_ref, w_ref, b_ref, o_ref):
    h = jnp.dot(x_ref[...], w_ref[...], preferred_element_type=jnp.float32)
    h = h + b_ref[...]
    o_ref[...] = jnp.concatenate(
        [h, jnp.zeros((BN_TC, IN_DIM - H), jnp.float32)], axis=1)


def _eproj_body(ea_ref, w_ref, b_ref, o1_ref, o2_ref, o3_ref):
    e = jnp.dot(ea_ref[...], w_ref[...],
                preferred_element_type=jnp.float32) + b_ref[...]
    o1_ref[...] = e[:, 0:128]
    o2_ref[...] = e[:, 128:256]
    o3_ref[...] = e[:, 256:384]


def _mlp_body(hp_ref, ag_ref, w1_ref, b1_ref, w2_ref, b2_ref, s_ref, t_ref,
              o_ref):
    ag = ag_ref[...]
    a0 = ag[0]
    a1 = ag[1]
    hin = hp_ref[...][:, :H] + jnp.concatenate([a0, a1], axis=1)
    t1 = jnp.maximum(
        jnp.dot(hin, w1_ref[...], preferred_element_type=jnp.float32)
        + b1_ref[...], 0.0)
    t2 = jnp.dot(t1, w2_ref[...], preferred_element_type=jnp.float32) + b2_ref[...]
    y = jnp.maximum(t2 * s_ref[...] + t_ref[...], 0.0)
    o_ref[...] = jnp.concatenate(
        [y, jnp.zeros((BN_TC, IN_DIM - H), jnp.float32)], axis=1)


def _head_body(sa_ref, sm_ref, w_ref, b_ref, g_ref, t_ref, o_ref):
    sa = jnp.sum(sa_ref[...], axis=0)
    mx = jnp.max(sm_ref[...], axis=0)
    sums = sa[:G, :H]
    cnt = sa[:G, H:H + 1]
    mean = sums / jnp.maximum(cnt, 1.0)
    cat = jnp.concatenate([mean, mx[:G, :H]], axis=1)
    o = jnp.dot(cat, w_ref[...], preferred_element_type=jnp.float32) + b_ref[...]
    mu = jnp.mean(o, axis=-1, keepdims=True)
    var = jnp.mean((o - mu) ** 2, axis=-1, keepdims=True)
    o = (o - mu) / jnp.sqrt(var + 1e-5) * g_ref[...] + t_ref[...]
    o_ref[...] = jnp.maximum(o, 0.0)


@functools.partial(
    pl.kernel,
    mesh=_mesh,
    out_type=jax.ShapeDtypeStruct((2, 2, HROWS, 128), jnp.float32),
    scratch_types=[
        pltpu.VMEM((16,), jnp.int32),
        pltpu.VMEM((16, 128), jnp.float32),
        pltpu.VMEM((2, 128), jnp.int32),
        pltpu.VMEM((2, 64), jnp.int32),
        pltpu.VMEM((2, 128), jnp.int32),
        pltpu.VMEM((2, 128), jnp.int32),
        pltpu.VMEM((2, 128, 128), jnp.float32),
        pltpu.VMEM((2, 64, 128), jnp.float32),
        pltpu.VMEM((128, 128), jnp.float32),
        pltpu.VMEM_SHARED((SROWS, 128), jnp.float32),
        pltpu.SemaphoreType.DMA,
        pltpu.SemaphoreType.DMA,
        pltpu.SemaphoreType.DMA,
        pltpu.SemaphoreType.DMA,
        pltpu.SemaphoreType.DMA,
    ],
)
def _edge_pass(hp, ep, out, sdidx, sdbuf, srcv, epidx, rowv, selv, hbuf, ebuf,
               vbuf, shared, semh0, semh1, seme0, seme1, sem3):
    cid = lax.axis_index("c")
    sid = lax.axis_index("s")
    z16 = jnp.zeros((16,), jnp.float32)
    iota = lax.iota(jnp.int32, 16)
    fo = cid * 32
    semh = (semh0, semh1)
    seme = (seme0, seme1)

    def zrow(i, _):
        for g in range(8):
            vbuf[i, pl.ds(g * 16, 16)] = z16
        return 0

    lax.fori_loop(0, 128, zrow, 0)

    def fetch_sd(g):
        sdidx[pl.ds(0, 16)] = (EOFF + g * 8
                               + jnp.where(iota < 8, iota, iota + SDROW - 8))
        pltpu.async_copy(ep.at[sdidx], sdbuf, sem3).wait()

    def prep_issue(ch, b, nbase):
        sc = ch % 8
        g = sid * NSUP + ch // 8
        ebase = g * 512 + sc * 64
        for k in range(4):
            epidx[b, pl.ds(k * 16, 16)] = ebase + k * 16 + iota
        for k in range(8):
            srcv[b, pl.ds(k * 16, 16)] = sdbuf[
                sc, pl.ds(k * 16, 16)].astype(jnp.int32)
            d = sdbuf[8 + sc, pl.ds(k * 16, 16)].astype(jnp.int32)
            dd = d - nbase
            valid = jnp.logical_and(dd >= 0, dd < HNODE)
            rowv[b, pl.ds(k * 16, 16)] = jnp.where(
                valid, jnp.right_shift(dd, 2), HROWS + 16)
            selv[b, pl.ds(k * 16, 16)] = jnp.bitwise_and(d, 3)
        pltpu.async_copy(hp.at[srcv.at[b]], hbuf.at[b], semh[b])
        pltpu.async_copy(ep.at[epidx.at[b]], ebuf.at[b], seme[b])

    def process(b):
        pltpu.make_async_copy(hp.at[srcv.at[b]], hbuf.at[b], semh[b]).wait()
        pltpu.make_async_copy(ep.at[epidx.at[b]], ebuf.at[b], seme[b]).wait()
        for j in range(128):
            sel = selv[b, pl.ds((j // 16) * 16, 16)][j % 16]
            co = sel * 32
            eb = (j % 2) * 64 + fo
            for gg in range(2):
                hv = hbuf[b, j, pl.ds(fo + gg * 16, 16)]
                ev = ebuf[b, j // 2, pl.ds(eb + gg * 16, 16)]
                vbuf[j, pl.ds(co + gg * 16, 16)] = jnp.maximum(hv + ev, 0.0)
        pltpu.sync_copy(vbuf, shared.at[rowv.at[b]], add=True)
        for j in range(128):
            sel = selv[b, pl.ds((j // 16) * 16, 16)][j % 16]
            co = sel * 32
            vbuf[j, pl.ds(co, 16)] = z16
            vbuf[j, pl.ds(co + 16, 16)] = z16

    def one_pass(p, _):
        nbase = p * HNODE
        for zo, zn in ((0, 112), (112, 112), (224, 112), (336, 64)):
            pltpu.sync_copy(vbuf.at[pl.ds(0, zn)],
                            shared.at[pl.ds(sid * 400 + zo, zn)])
        plsc.subcore_barrier()

        fetch_sd(sid * NSUP)
        prep_issue(0, 0, nbase)

        def pair(i2, _):
            ch = i2 * 2
            prep_issue(ch + 1, 1, nbase)
            process(0)

            @pl.when(i2 + 1 < NSUP * 4)
            def _():
                @pl.when((ch + 2) % 8 == 0)
                def _():
                    fetch_sd(sid * NSUP + (ch + 2) // 8)
                prep_issue(ch + 2, 0, nbase)

            process(1)
            return 0

        lax.fori_loop(0, NSUP * 4, pair, 0)
        plsc.subcore_barrier()

        for zo, zn in ((0, 112), (112, 112), (224, 112), (336, 56)):
            off = sid * 392 + zo
            pltpu.sync_copy(shared.at[pl.ds(off, zn)],
                            out.at[cid, p, pl.ds(off, zn)])
        plsc.subcore_barrier()
        return 0

    lax.fori_loop(0, 2, one_pass, 0)


@functools.partial(
    pl.kernel,
    mesh=_mesh,
    out_type=(
        jax.ShapeDtypeStruct((32, 72, 128), jnp.float32),
        jax.ShapeDtypeStruct((32, 72, 128), jnp.float32),
    ),
    scratch_types=[
        pltpu.VMEM((PCH,), jnp.int32),
        pltpu.VMEM((PCH, 128), jnp.float32),
        pltpu.VMEM((72, 128), jnp.float32),
        pltpu.VMEM((72, 128), jnp.float32),
        pltpu.SemaphoreType.DMA,
    ],
)
def _pool(hp, batch, out_a, out_m, bv, hbuf, acc_a, acc_m, sem):
    cid = lax.axis_index("c")
    sid = lax.axis_index("s")
    wid = sid * 2 + cid
    z16 = jnp.zeros((16,), jnp.float32)
    ninf = jnp.full((16,), -jnp.inf, jnp.float32)
    one0 = jnp.where(lax.iota(jnp.int32, 16) == 0, 1.0, 0.0).astype(jnp.float32)

    def zrow(i, _):
        for g in range(8):
            acc_a[i, pl.ds(g * 16, 16)] = z16
            acc_m[i, pl.ds(g * 16, 16)] = ninf
        return 0

    lax.fori_loop(0, 72, zrow, 0)

    base = wid * PT

    def chunk(ci, _):
        off = base + ci * PCH
        pltpu.sync_copy(batch.at[pl.ds(off, PCH)], bv)
        pltpu.sync_copy(hp.at[pl.ds(off, PCH)], hbuf)
        for j in range(PCH):
            b = bv[pl.ds((j // 16) * 16, 16)][j % 16]
            for g in range(4):
                s = pl.ds(g * 16, 16)
                acc_a[b, s] += hbuf[j, s]
                acc_m[b, s] = jnp.maximum(acc_m[b, s], hbuf[j, s])
            acc_a[b, pl.ds(64, 16)] += one0
        return 0

    lax.fori_loop(0, NPCH, chunk, 0)
    pltpu.sync_copy(acc_a, out_a.at[wid])
    pltpu.sync_copy(acc_m, out_m.at[wid])


def _tc_call(body, grid, in_specs, out_specs, out_shape):
    return pl.pallas_call(
        body, grid=grid, in_specs=in_specs, out_specs=out_specs,
        out_shape=out_shape)


def kernel(x, edge_index, edge_attr, batch, np_W, np_b, lin1_W, lin1_b,
           mlp1_W1, mlp1_b1, mlp1_W2, mlp1_b2, bn1_g, bn1_b, lin2_W, lin2_b,
           mlp2_W1, mlp2_b1, mlp2_W2, mlp2_b2, bn2_g, bn2_b, lin3_W, lin3_b,
           mlp3_W1, mlp3_b1, mlp3_W2, mlp3_b2, bn3_g, bn3_b, proj_W, proj_b,
           ln_g, ln_b):
    f32 = jnp.float32
    srcp = jnp.concatenate(
        [edge_index[0], jnp.zeros((E2 - E,), jnp.int32)])
    dstp = jnp.concatenate(
        [edge_index[1], jnp.full((E2 - E,), NP - 1, jnp.int32)])
    sd_rows = jnp.concatenate(
        [srcp, dstp]).reshape(2 * SDROW, 128).astype(jnp.float32)
    x_pad = jnp.zeros((NP, IN_DIM), f32).at[:N].set(x)
    batch_pad = jnp.concatenate(
        [batch, jnp.full((NP - N,), G, jnp.int32)])
    ea32 = edge_attr.reshape(E // 2, 2 * ED)

    wfull = lambda shp: pl.BlockSpec(shp, lambda i: (0,) * len(shp))

    hp = _tc_call(
        _prep_body, (NBLK,),
        [pl.BlockSpec((BN_TC, IN_DIM), lambda i: (i, 0)),
         wfull((IN_DIM, H)), wfull((1, H))],
        pl.BlockSpec((BN_TC, IN_DIM), lambda i: (i, 0)),
        jax.ShapeDtypeStruct((NP, IN_DIM), f32),
    )(x_pad, np_W, np_b.reshape(1, H))

    zw = jnp.zeros((ED, H), f32)
    wbd = jnp.block([
        [lin1_W, zw, lin2_W, zw, lin3_W, zw],
        [zw, lin1_W, zw, lin2_W, zw, lin3_W],
    ])
    bbd = jnp.concatenate(
        [lin1_b, lin1_b, lin2_b, lin2_b, lin3_b, lin3_b]).reshape(1, 6 * H)
    eshape = jax.ShapeDtypeStruct((E // 2, 128), f32)
    e1, e2, e3 = _tc_call(
        _eproj_body, (E // 8000,),
        [pl.BlockSpec((4000, 2 * ED), lambda i: (i, 0)),
         wfull((2 * ED, 6 * H)), wfull((1, 6 * H))],
        [pl.BlockSpec((4000, 128), lambda i: (i, 0))] * 3,
        [eshape, eshape, eshape],
    )(ea32, wbd, bbd)

    mlp_in_specs = [
        pl.BlockSpec((BN_TC, IN_DIM), lambda i: (i, 0)),
        pl.BlockSpec((2, BN_TC, 32), lambda i: (0, i, 0)),
        wfull((H, H)), wfull((1, H)), wfull((H, H)), wfull((1, H)),
        wfull((1, H)), wfull((1, H)),
    ]
    bn_s = 1.0 / jnp.sqrt(jnp.float32(1.0 + 1e-5))

    for (ep, w1, b1, w2, b2, bg, bb) in (
        (e1, mlp1_W1, mlp1_b1, mlp1_W2, mlp1_b2, bn1_g, bn1_b),
        (e2, mlp2_W1, mlp2_b1, mlp2_W2, mlp2_b2, bn2_g, bn2_b),
        (e3, mlp3_W1, mlp3_b1, mlp3_W2, mlp3_b2, bn3_g, bn3_b),
    ):
        combo = jnp.concatenate(
            [ep, jnp.zeros(((E2 - E) // 2, 128), f32), sd_rows])
        aggr = _edge_pass(hp, combo).reshape(2, NP, 32)
        hp = _tc_call(
            _mlp_body, (NBLK,),
            mlp_in_specs,
            pl.BlockSpec((BN_TC, IN_DIM), lambda i: (i, 0)),
            jax.ShapeDtypeStruct((NP, IN_DIM), f32),
        )(hp, aggr, w1, b1.reshape(1, H), w2, b2.reshape(1, H),
          (bg * bn_s).reshape(1, H), bb.reshape(1, H))

    sa, sm = _pool(hp, batch_pad)

    out = _tc_call(
        _head_body, (1,),
        [wfull((32, 72, 128)), wfull((32, 72, 128)),
         wfull((2 * H, EMB)), wfull((1, EMB)), wfull((1, EMB)),
         wfull((1, EMB))],
        wfull((G, EMB)),
        jax.ShapeDtypeStruct((G, EMB), f32),
    )(sa, sm, proj_W, proj_b.reshape(1, EMB), ln_g.reshape(1, EMB),
      ln_b.reshape(1, EMB))
    return out
